# baseline (speedup 1.0000x reference)
import functools

import jax
import jax.numpy as jnp
from jax import lax
from jax.experimental import pallas as pl
from jax.experimental.pallas import tpu as pltpu
from jax.experimental.pallas import tpu_sc as plsc

B, H, W, C = 8, 224, 224, 192
N = B * H * W
HW = H * W
L = 16
NW = 32
PIX_PER_W = N // NW
K = 32
NCHUNKS = PIX_PER_W // K
CB = C // L
X = N * C // 128


def _floor_f32(x):
    t = x.astype(jnp.int32).astype(jnp.float32)
    return jnp.where(t > x, t - 1.0, t)


def _splat(ref, i):
    return plsc.load_gather(ref, [jnp.full((L,), i, jnp.int32)])


def _sc_body(img_hbm, theta_hbm, xt_hbm, yt_hbm, out_hbm, theta_v, xt_v,
             yt_v, idx_v, w_v, rows_v, out_v, sem):
    wid = lax.axis_index("s") * 2 + lax.axis_index("c")
    base = wid * PIX_PER_W
    b = wid // (NW // B)
    bhw = b * HW

    pltpu.sync_copy(theta_hbm, theta_v.at[pl.ds(0, B * 6)])
    pltpu.sync_copy(xt_hbm, xt_v)
    pltpu.sync_copy(yt_hbm, yt_v)
    t = [_splat(theta_v, b * 6 + j) for j in range(6)]

    iota = lax.iota(jnp.int32, L)

    def chunk(g, carry):
        off = base + g * K
        for u in range(K // L):
            p = off + u * L + iota
            rr = p - bhw
            hh = rr // W
            ww = rr - hh * W
            xt = plsc.load_gather(xt_v, [ww])
            yt = plsc.load_gather(yt_v, [hh])
            xs = (t[0] * xt + t[1] * yt) + t[2]
            ys = (t[3] * xt + t[4] * yt) + t[5]
            xf = 0.5 * ((xs + 1.0) * float(W) - 1.0)
            yf = 0.5 * ((ys + 1.0) * float(H) - 1.0)
            xf = jnp.minimum(jnp.maximum(xf, -4.0), float(W) + 4.0)
            yf = jnp.minimum(jnp.maximum(yf, -4.0), float(H) + 4.0)
            x0f = _floor_f32(xf)
            y0f = _floor_f32(yf)
            omfx = (x0f + 1.0) - xf
            omfy = (y0f + 1.0) - yf
            fx = xf - x0f
            fy = yf - y0f
            x0i = x0f.astype(jnp.int32)
            y0i = y0f.astype(jnp.int32)
            x0c = jnp.minimum(jnp.maximum(x0i, 0), W - 1)
            x1c = jnp.minimum(jnp.maximum(x0i + 1, 0), W - 1)
            y0c = jnp.minimum(jnp.maximum(y0i, 0), H - 1)
            y1c = jnp.minimum(jnp.maximum(y0i + 1, 0), H - 1)
            par0 = (x0c & 1).astype(jnp.float32)
            par1 = (x1c & 1).astype(jnp.float32)
            wgt = [omfx * omfy, omfx * fy, fx * omfy, fx * fy]
            par = [par0, par0, par1, par1]
            r0 = bhw + y0c * W
            r1 = bhw + y1c * W
            rn = [r0 + x0c, r1 + x0c, r0 + x1c, r1 + x1c]
            pos = 2 * (u * L + iota)
            for n in range(4):
                a = lax.shift_right_logical(3 * rn[n], 1)
                nn = jnp.full((L,), n, jnp.int32)
                plsc.store_scatter(idx_v, [nn, pos], a)
                plsc.store_scatter(idx_v, [nn, pos + 1], a + 1)
                w_v[pl.ds((2 * n) * K + u * L, L)] = wgt[n] * (1.0 - par[n])
                w_v[pl.ds((2 * n + 1) * K + u * L, L)] = wgt[n] * par[n]
        cps = [pltpu.async_copy(img_hbm.at[idx_v.at[n]], rows_v.at[n], sem)
               for n in range(4)]
        for cp in cps:
            cp.wait()

        def blend(k, carry2):
            ws = [(_splat(w_v, (2 * n) * K + k),
                   _splat(w_v, (2 * n + 1) * K + k)) for n in range(4)]
            for cb in range(CB):
                s0 = 1 if cb >= 8 else 0
                o0 = cb * L - (128 if cb >= 8 else 0)
                s1 = 1 if cb >= 4 else 0
                o1 = 64 + cb * L - (128 if cb >= 4 else 0)
                acc = None
                for n in range(4):
                    w0, w1 = ws[n]
                    t0 = w0 * rows_v[n, 2 * k + s0, pl.ds(o0, L)]
                    acc = t0 if acc is None else acc + t0
                    acc = acc + w1 * rows_v[n, 2 * k + s1, pl.ds(o1, L)]
                out_v[pl.ds(k * C + cb * L, L)] = acc
            return carry2

        lax.fori_loop(0, K, blend, 0)
        pltpu.sync_copy(out_v, out_hbm.at[pl.ds(off * C, K * C)])
        return carry

    lax.fori_loop(0, NCHUNKS, chunk, 0)


@jax.jit
def _spatial_transform(table, theta_b, xt_b, yt_b):
    mesh = plsc.VectorSubcoreMesh(core_axis_name="c", subcore_axis_name="s")
    f = functools.partial(
        pl.kernel,
        mesh=mesh,
        compiler_params=pltpu.CompilerParams(needs_layout_passes=False,
                                             use_tc_tiling_on_sc=True),
        out_type=jax.ShapeDtypeStruct((N * C,), jnp.float32),
        scratch_types=[
            pltpu.VMEM((B * 6 + L,), jnp.float32),
            pltpu.VMEM((W,), jnp.float32),
            pltpu.VMEM((H,), jnp.float32),
            pltpu.VMEM((4, 2 * K), jnp.int32),
            pltpu.VMEM((8 * K,), jnp.float32),
            pltpu.VMEM((4, 2 * K, 128), jnp.float32),
            pltpu.VMEM((K * C,), jnp.float32),
            pltpu.SemaphoreType.DMA,
        ],
    )(_sc_body)
    return f(table, theta_b, xt_b, yt_b)


def kernel(images, theta):
    table = images.reshape(X, 128)
    theta_b = lax.optimization_barrier(
        theta.astype(jnp.bfloat16)).astype(jnp.float32).reshape(B * 6)
    xt_b = lax.optimization_barrier(
        jnp.linspace(-1.0, 1.0, W).astype(jnp.float32)
        .astype(jnp.bfloat16)).astype(jnp.float32)
    yt_b = lax.optimization_barrier(
        jnp.linspace(-1.0, 1.0, H).astype(jnp.float32)
        .astype(jnp.bfloat16)).astype(jnp.float32)
    out = _spatial_transform(table, theta_b, xt_b, yt_b)
    return out.reshape(B, H, W, C)


# speedup vs baseline: 1.5361x; 1.5361x over previous
import functools

import jax
import jax.numpy as jnp
from jax import lax
from jax.experimental import pallas as pl
from jax.experimental.pallas import tpu as pltpu
from jax.experimental.pallas import tpu_sc as plsc

B, H, W, C = 8, 224, 224, 192
N = B * H * W
HW = H * W
L = 16
NW = 32
PIX_PER_W = N // NW
K = 64
NCHUNKS = PIX_PER_W // K
CB = C // L


def _floor_f32(x):
    t = x.astype(jnp.int32).astype(jnp.float32)
    return jnp.where(t > x, t - 1.0, t)


def _splat(ref, i):
    return plsc.load_gather(ref, [jnp.full((L,), i, jnp.int32)])


def _sc_body(img_hbm, theta_hbm, xt_hbm, yt_hbm, out_hbm, theta_v, xt_v,
             yt_v, idx_v, w_v, rows_v, out_v, sem0, sem1):
    wid = lax.axis_index("s") * 2 + lax.axis_index("c")
    base = wid * PIX_PER_W
    b = wid // (NW // B)
    bhw = b * HW

    pltpu.sync_copy(theta_hbm, theta_v.at[pl.ds(0, B * 6)])
    pltpu.sync_copy(xt_hbm, xt_v)
    pltpu.sync_copy(yt_hbm, yt_v)
    t = [_splat(theta_v, b * 6 + j) for j in range(6)]

    iota = lax.iota(jnp.int32, L)
    sems = (sem0, sem1)

    def do_idx(g, bf):
        off = base + g * K
        for u in range(K // L):
            p = off + u * L + iota
            rr = p - bhw
            hh = rr // W
            ww = rr - hh * W
            xt = plsc.load_gather(xt_v, [ww])
            yt = plsc.load_gather(yt_v, [hh])
            xs = (t[0] * xt + t[1] * yt) + t[2]
            ys = (t[3] * xt + t[4] * yt) + t[5]
            xf = 0.5 * ((xs + 1.0) * float(W) - 1.0)
            yf = 0.5 * ((ys + 1.0) * float(H) - 1.0)
            xf = jnp.minimum(jnp.maximum(xf, -4.0), float(W) + 4.0)
            yf = jnp.minimum(jnp.maximum(yf, -4.0), float(H) + 4.0)
            x0f = _floor_f32(xf)
            y0f = _floor_f32(yf)
            omfx = (x0f + 1.0) - xf
            omfy = (y0f + 1.0) - yf
            fx = xf - x0f
            fy = yf - y0f
            wb0 = bf * 4 * K
            w_v[pl.ds(wb0 + 0 * K + u * L, L)] = omfx * omfy
            w_v[pl.ds(wb0 + 1 * K + u * L, L)] = omfx * fy
            w_v[pl.ds(wb0 + 2 * K + u * L, L)] = fx * omfy
            w_v[pl.ds(wb0 + 3 * K + u * L, L)] = fx * fy
            x0i = x0f.astype(jnp.int32)
            y0i = y0f.astype(jnp.int32)
            x0c = jnp.minimum(jnp.maximum(x0i, 0), W - 1)
            x1c = jnp.minimum(jnp.maximum(x0i + 1, 0), W - 1)
            y0c = jnp.minimum(jnp.maximum(y0i, 0), H - 1)
            y1c = jnp.minimum(jnp.maximum(y0i + 1, 0), H - 1)
            r0 = bhw + y0c * W
            r1 = bhw + y1c * W
            idx_v[bf, 0, pl.ds(u * L, L)] = r0 + x0c
            idx_v[bf, 0, pl.ds(K + u * L, L)] = r1 + x0c
            idx_v[bf, 1, pl.ds(u * L, L)] = r0 + x1c
            idx_v[bf, 1, pl.ds(K + u * L, L)] = r1 + x1c

    def fire(bf):
        pltpu.async_copy(img_hbm.at[idx_v.at[bf, 0]], rows_v.at[bf, 0],
                         sems[bf])
        pltpu.async_copy(img_hbm.at[idx_v.at[bf, 1]], rows_v.at[bf, 1],
                         sems[bf])

    def drain(bf):
        pltpu.make_async_copy(img_hbm.at[idx_v.at[bf, 0]], rows_v.at[bf, 0],
                              sems[bf]).wait()
        pltpu.make_async_copy(img_hbm.at[idx_v.at[bf, 1]], rows_v.at[bf, 1],
                              sems[bf]).wait()

    def blend_out(g, bf):
        off = base + g * K

        def blend(k, carry2):
            wb0 = bf * 4 * K
            wa = _splat(w_v, wb0 + k)
            wb = _splat(w_v, wb0 + k + K)
            wc = _splat(w_v, wb0 + k + 2 * K)
            wd = _splat(w_v, wb0 + k + 3 * K)
            for cb in range(CB):
                s = pl.ds(cb * L, L)
                acc = wa * rows_v[bf, 0, k, s]
                acc = acc + wb * rows_v[bf, 0, K + k, s]
                acc = acc + wc * rows_v[bf, 1, k, s]
                acc = acc + wd * rows_v[bf, 1, K + k, s]
                out_v[pl.ds(k * C + cb * L, L)] = acc
            return carry2

        lax.fori_loop(0, K, blend, 0)
        pltpu.sync_copy(out_v, out_hbm.at[pl.ds(off * C, K * C)])

    do_idx(0, 0)
    fire(0)

    def step(i, carry):
        g0 = 2 * i
        do_idx(g0 + 1, 1)
        fire(1)
        drain(0)
        blend_out(g0, 0)

        @pl.when(i < NCHUNKS // 2 - 1)
        def _():
            do_idx(g0 + 2, 0)
            fire(0)

        drain(1)
        blend_out(g0 + 1, 1)
        return carry

    lax.fori_loop(0, NCHUNKS // 2, step, 0)


@jax.jit
def _spatial_transform(flat_images, theta_b, xt_b, yt_b):
    mesh = plsc.VectorSubcoreMesh(core_axis_name="c", subcore_axis_name="s")
    f = functools.partial(
        pl.kernel,
        mesh=mesh,
        compiler_params=pltpu.CompilerParams(needs_layout_passes=False,
                                             use_tc_tiling_on_sc=False),
        out_type=jax.ShapeDtypeStruct((N * C,), jnp.float32),
        scratch_types=[
            pltpu.VMEM((B * 6 + L,), jnp.float32),
            pltpu.VMEM((W,), jnp.float32),
            pltpu.VMEM((H,), jnp.float32),
            pltpu.VMEM((2, 2, 2 * K), jnp.int32),
            pltpu.VMEM((8 * K,), jnp.float32),
            pltpu.VMEM((2, 2, 2 * K, C), jnp.float32),
            pltpu.VMEM((K * C,), jnp.float32),
            pltpu.SemaphoreType.DMA,
            pltpu.SemaphoreType.DMA,
        ],
    )(_sc_body)
    return f(flat_images, theta_b, xt_b, yt_b)


def kernel(images, theta):
    flat = images.reshape(N, C)
    theta_b = lax.optimization_barrier(
        theta.astype(jnp.bfloat16)).astype(jnp.float32).reshape(B * 6)
    xt_b = lax.optimization_barrier(
        jnp.linspace(-1.0, 1.0, W).astype(jnp.float32)
        .astype(jnp.bfloat16)).astype(jnp.float32)
    yt_b = lax.optimization_barrier(
        jnp.linspace(-1.0, 1.0, H).astype(jnp.float32)
        .astype(jnp.bfloat16)).astype(jnp.float32)
    out = _spatial_transform(flat, theta_b, xt_b, yt_b)
    return out.reshape(B, H, W, C)


# speedup vs baseline: 1.5481x; 1.0078x over previous
import functools

import jax
import jax.numpy as jnp
from jax import lax
from jax.experimental import pallas as pl
from jax.experimental.pallas import tpu as pltpu
from jax.experimental.pallas import tpu_sc as plsc

B, H, W, C = 8, 224, 224, 192
N = B * H * W
HW = H * W
L = 16
NW = 32
PIX_PER_W = N // NW
K = 64
NCHUNKS = PIX_PER_W // K
CB = C // L


def _floor_f32(x):
    t = x.astype(jnp.int32).astype(jnp.float32)
    return jnp.where(t > x, t - 1.0, t)


def _splat(ref, i):
    return plsc.load_gather(ref, [jnp.full((L,), i, jnp.int32)])


def _sc_body(img_hbm, theta_hbm, xt_hbm, yt_hbm, out_hbm, theta_v, xt_v,
             yt_v, idx_v, w_v, rows_v, out_v, sem0, sem1, osem0, osem1):
    wid = lax.axis_index("s") * 2 + lax.axis_index("c")
    base = wid * PIX_PER_W
    b = wid // (NW // B)
    bhw = b * HW

    pltpu.sync_copy(theta_hbm, theta_v.at[pl.ds(0, B * 6)])
    pltpu.sync_copy(xt_hbm, xt_v)
    pltpu.sync_copy(yt_hbm, yt_v)
    t = [_splat(theta_v, b * 6 + j) for j in range(6)]

    iota = lax.iota(jnp.int32, L)
    sems = (sem0, sem1)
    osems = (osem0, osem1)

    def do_idx(g, bf):
        off = base + g * K
        for u in range(K // L):
            p = off + u * L + iota
            rr = p - bhw
            hh = rr // W
            ww = rr - hh * W
            xt = plsc.load_gather(xt_v, [ww])
            yt = plsc.load_gather(yt_v, [hh])
            xs = (t[0] * xt + t[1] * yt) + t[2]
            ys = (t[3] * xt + t[4] * yt) + t[5]
            xf = 0.5 * ((xs + 1.0) * float(W) - 1.0)
            yf = 0.5 * ((ys + 1.0) * float(H) - 1.0)
            xf = jnp.minimum(jnp.maximum(xf, -4.0), float(W) + 4.0)
            yf = jnp.minimum(jnp.maximum(yf, -4.0), float(H) + 4.0)
            x0f = _floor_f32(xf)
            y0f = _floor_f32(yf)
            omfx = (x0f + 1.0) - xf
            omfy = (y0f + 1.0) - yf
            fx = xf - x0f
            fy = yf - y0f
            wb0 = bf * 4 * K
            w_v[pl.ds(wb0 + 0 * K + u * L, L)] = omfx * omfy
            w_v[pl.ds(wb0 + 1 * K + u * L, L)] = omfx * fy
            w_v[pl.ds(wb0 + 2 * K + u * L, L)] = fx * omfy
            w_v[pl.ds(wb0 + 3 * K + u * L, L)] = fx * fy
            x0i = x0f.astype(jnp.int32)
            y0i = y0f.astype(jnp.int32)
            x0c = jnp.minimum(jnp.maximum(x0i, 0), W - 1)
            x1c = jnp.minimum(jnp.maximum(x0i + 1, 0), W - 1)
            y0c = jnp.minimum(jnp.maximum(y0i, 0), H - 1)
            y1c = jnp.minimum(jnp.maximum(y0i + 1, 0), H - 1)
            r0 = bhw + y0c * W
            r1 = bhw + y1c * W
            idx_v[bf, 0, pl.ds(u * L, L)] = r0 + x0c
            idx_v[bf, 0, pl.ds(K + u * L, L)] = r1 + x0c
            idx_v[bf, 1, pl.ds(u * L, L)] = r0 + x1c
            idx_v[bf, 1, pl.ds(K + u * L, L)] = r1 + x1c

    def fire(bf):
        pltpu.async_copy(img_hbm.at[idx_v.at[bf, 0]], rows_v.at[bf, 0],
                         sems[bf])
        pltpu.async_copy(img_hbm.at[idx_v.at[bf, 1]], rows_v.at[bf, 1],
                         sems[bf])

    def drain(bf):
        pltpu.make_async_copy(img_hbm.at[idx_v.at[bf, 0]], rows_v.at[bf, 0],
                              sems[bf]).wait()
        pltpu.make_async_copy(img_hbm.at[idx_v.at[bf, 1]], rows_v.at[bf, 1],
                              sems[bf]).wait()

    def blend_out(g, bf, first):
        off = base + g * K

        def blend(k, carry2):
            wb0 = bf * 4 * K
            wa = _splat(w_v, wb0 + k)
            wb = _splat(w_v, wb0 + k + K)
            wc = _splat(w_v, wb0 + k + 2 * K)
            wd = _splat(w_v, wb0 + k + 3 * K)
            for cb in range(CB):
                s = pl.ds(cb * L, L)
                acc = wa * rows_v[bf, 0, k, s]
                acc = acc + wb * rows_v[bf, 0, K + k, s]
                acc = acc + wc * rows_v[bf, 1, k, s]
                acc = acc + wd * rows_v[bf, 1, K + k, s]
                out_v[bf, pl.ds(k * C + cb * L, L)] = acc
            return carry2

        @pl.when(jnp.logical_not(first))
        def _():
            pltpu.make_async_copy(
                out_v.at[bf], out_hbm.at[pl.ds(0, K * C)], osems[bf]).wait()

        lax.fori_loop(0, K, blend, 0)
        pltpu.async_copy(out_v.at[bf], out_hbm.at[pl.ds(off * C, K * C)],
                         osems[bf])

    do_idx(0, 0)
    fire(0)

    def step(i, carry):
        g0 = 2 * i
        do_idx(g0 + 1, 1)
        fire(1)
        drain(0)
        blend_out(g0, 0, i == 0)

        @pl.when(i < NCHUNKS // 2 - 1)
        def _():
            do_idx(g0 + 2, 0)
            fire(0)

        drain(1)
        blend_out(g0 + 1, 1, i == 0)
        return carry

    lax.fori_loop(0, NCHUNKS // 2, step, 0)
    for bf in range(2):
        pltpu.make_async_copy(out_v.at[bf], out_hbm.at[pl.ds(0, K * C)],
                              osems[bf]).wait()


@jax.jit
def _spatial_transform(flat_images, theta_b, xt_b, yt_b):
    mesh = plsc.VectorSubcoreMesh(core_axis_name="c", subcore_axis_name="s")
    f = functools.partial(
        pl.kernel,
        mesh=mesh,
        compiler_params=pltpu.CompilerParams(needs_layout_passes=False,
                                             use_tc_tiling_on_sc=False),
        out_type=jax.ShapeDtypeStruct((N * C,), jnp.float32),
        scratch_types=[
            pltpu.VMEM((B * 6 + L,), jnp.float32),
            pltpu.VMEM((W,), jnp.float32),
            pltpu.VMEM((H,), jnp.float32),
            pltpu.VMEM((2, 2, 2 * K), jnp.int32),
            pltpu.VMEM((8 * K,), jnp.float32),
            pltpu.VMEM((2, 2, 2 * K, C), jnp.float32),
            pltpu.VMEM((2, K * C), jnp.float32),
            pltpu.SemaphoreType.DMA,
            pltpu.SemaphoreType.DMA,
            pltpu.SemaphoreType.DMA,
            pltpu.SemaphoreType.DMA,
        ],
    )(_sc_body)
    return f(flat_images, theta_b, xt_b, yt_b)


def kernel(images, theta):
    flat = images.reshape(N, C)
    theta_b = lax.optimization_barrier(
        theta.astype(jnp.bfloat16)).astype(jnp.float32).reshape(B * 6)
    xt_b = lax.optimization_barrier(
        jnp.linspace(-1.0, 1.0, W).astype(jnp.float32)
        .astype(jnp.bfloat16)).astype(jnp.float32)
    yt_b = lax.optimization_barrier(
        jnp.linspace(-1.0, 1.0, H).astype(jnp.float32)
        .astype(jnp.bfloat16)).astype(jnp.float32)
    out = _spatial_transform(flat, theta_b, xt_b, yt_b)
    return out.reshape(B, H, W, C)
